# Initial kernel scaffold; baseline (speedup 1.0000x reference)
import jax
import jax.numpy as jnp
from jax.experimental import pallas as pl


def kernel(x1, x2, x3, mask, device, emb1, emb2, emb3, W1, b1, W2, b2, W3, b3):
    raise NotImplementedError("write your pallas kernel here")


# speedup vs baseline: 1.4080x; 1.4080x over previous
import functools

import jax
import jax.numpy as jnp
from jax import lax
from jax.experimental import pallas as pl
from jax.experimental.pallas import tpu as pltpu
from jax.experimental.pallas import tpu_sc as plsc

_B = 16384
_L = 50
_V1, _V2, _V3 = 101, 101, 49
_E1, _E2, _E3 = 96, 96, 64
_TE = _E1 + _E2 + _E3
_MD = 256

_NC, _NS = 2, 16
_NW = _NC * _NS
_RPW = _B // _NW
_CB = 32
_NCH = _RPW // _CB


def _tables_body(w_ref, e1_ref, e2_ref, e3_ref, t1_ref, t2_ref, t3_ref):
    w = w_ref[0]
    dn = (((1,), (1,)), ((), ()))
    t1_ref[0] = lax.dot_general(e1_ref[...], w[:, 0:_E1], dn,
                                preferred_element_type=jnp.float32)
    t2_ref[0] = lax.dot_general(e2_ref[...], w[:, _E1:_E1 + _E2], dn,
                                preferred_element_type=jnp.float32)
    t3_ref[0] = lax.dot_general(e3_ref[...], w[:, _E1 + _E2:_TE], dn,
                                preferred_element_type=jnp.float32)


def _make_tables(W1, emb1, emb2, emb3):
    w1r = W1.reshape(_MD, _L, _TE).transpose(1, 0, 2)
    t1, t2, t3 = pl.pallas_call(
        _tables_body,
        grid=(_L,),
        in_specs=[
            pl.BlockSpec((1, _MD, _TE), lambda l: (l, 0, 0)),
            pl.BlockSpec((_V1, _E1), lambda l: (0, 0)),
            pl.BlockSpec((_V2, _E2), lambda l: (0, 0)),
            pl.BlockSpec((_V3, _E3), lambda l: (0, 0)),
        ],
        out_specs=[
            pl.BlockSpec((1, _V1, _MD), lambda l: (l, 0, 0)),
            pl.BlockSpec((1, _V2, _MD), lambda l: (l, 0, 0)),
            pl.BlockSpec((1, _V3, _MD), lambda l: (l, 0, 0)),
        ],
        out_shape=[
            jax.ShapeDtypeStruct((_L, _V1, _MD), jnp.float32),
            jax.ShapeDtypeStruct((_L, _V2, _MD), jnp.float32),
            jax.ShapeDtypeStruct((_L, _V3, _MD), jnp.float32),
        ],
    )(w1r, emb1, emb2, emb3)
    return (t1.reshape(_L * _V1, _MD),
            t2.reshape(_L * _V2, _MD),
            t3.reshape(_L * _V3, _MD))


_NI = 160
_GH = _NI // 2


def _bag_body(idx_h, tf_h, out_h, idx_v, gbuf, obuf, sem_g, sem_o):
    wid = lax.axis_index("s") * _NC + lax.axis_index("c")
    base = wid * _RPW

    def chunk_body(ch, carry):
        cbase = base + ch * _CB
        pltpu.sync_copy(idx_h.at[pl.ds(cbase * _NI, _CB * _NI)], idx_v)

        def row_body(r, carry2):
            c1 = pltpu.async_copy(tf_h.at[idx_v.at[pl.ds(r * _NI, _GH)]],
                                  gbuf.at[0, pl.ds(0, _GH)], sem_g)
            c2 = pltpu.async_copy(tf_h.at[idx_v.at[pl.ds(r * _NI + _GH, _GH)]],
                                  gbuf.at[0, pl.ds(_GH, _GH)], sem_g)
            c1.wait()
            c2.wait()

            def acc_body(j, acc):
                return tuple(acc[k] + gbuf[0, j, pl.ds(k * 16, 16)]
                             for k in range(16))

            acc0 = tuple(gbuf[0, 0, pl.ds(k * 16, 16)] for k in range(16))
            acc = lax.fori_loop(1, _NI, acc_body, acc0)
            for k in range(16):
                obuf[r, pl.ds(k * 16, 16)] = acc[k]
            return carry2

        lax.fori_loop(0, _CB, row_body, 0)
        co = pltpu.async_copy(obuf, out_h.at[pl.ds(cbase, _CB)], sem_o)
        co.wait()
        return carry

    lax.fori_loop(0, _NCH, chunk_body, 0)


def _bag(idx, tf):
    mesh = plsc.VectorSubcoreMesh(core_axis_name="c", subcore_axis_name="s",
                                  num_cores=_NC, num_subcores=_NS)
    return pl.kernel(
        _bag_body,
        out_type=jax.ShapeDtypeStruct((_B, _MD), jnp.float32),
        mesh=mesh,
        scratch_types=[
            pltpu.VMEM((_CB * _NI,), jnp.int32),
            pltpu.VMEM((2, _NI, _MD), jnp.float32),
            pltpu.VMEM((_CB, _MD), jnp.float32),
            pltpu.SemaphoreType.DMA,
            pltpu.SemaphoreType.DMA,
        ],
    )(idx, tf)


_MLP_BLK = 1024


def _mlp_body(h_ref, b1_ref, w2_ref, b2_ref, w3_ref, b3_ref, o_ref):
    dn = (((1,), (1,)), ((), ()))
    x = jnp.maximum(h_ref[...] + b1_ref[...], 0.0)
    x = lax.dot_general(x, w2_ref[...], dn,
                        preferred_element_type=jnp.float32) + b2_ref[...]
    x = jnp.maximum(x, 0.0)
    o = lax.dot_general(x, w3_ref[...], dn,
                        preferred_element_type=jnp.float32) + b3_ref[0, 0]
    o_ref[...] = o[:, 0:1]


def _mlp(h1, b1, W2, b2, W3, b3):
    return pl.pallas_call(
        _mlp_body,
        grid=(_B // _MLP_BLK,),
        in_specs=[
            pl.BlockSpec((_MLP_BLK, _MD), lambda i: (i, 0)),
            pl.BlockSpec((1, _MD), lambda i: (0, 0)),
            pl.BlockSpec((_MD, _MD), lambda i: (0, 0)),
            pl.BlockSpec((1, _MD), lambda i: (0, 0)),
            pl.BlockSpec((8, _MD), lambda i: (0, 0)),
            pl.BlockSpec((1, 1), lambda i: (0, 0)),
        ],
        out_specs=pl.BlockSpec((_MLP_BLK, 1), lambda i: (i, 0)),
        out_shape=jax.ShapeDtypeStruct((_B, 1), jnp.float32),
    )(h1, b1.reshape(1, _MD), W2, b2.reshape(1, _MD),
      jnp.pad(W3, ((0, 7), (0, 0))), b3.reshape(1, 1))


def kernel(x1, x2, x3, mask, device, emb1, emb2, emb3,
           W1, b1, W2, b2, W3, b3):
    del mask, device
    t1f, t2f, t3f = _make_tables(W1, emb1, emb2, emb3)
    nrows = _L * (_V1 + _V2 + _V3)
    tf = jnp.concatenate(
        [t1f, t2f, t3f, jnp.zeros((_NI - 150 + 6, _MD), jnp.float32)], axis=0)
    pos1 = (jnp.arange(_L, dtype=jnp.int32) * _V1)[None, :]
    pos2 = (jnp.arange(_L, dtype=jnp.int32) * _V2)[None, :]
    pos3 = (jnp.arange(_L, dtype=jnp.int32) * _V3)[None, :]
    idx = jnp.concatenate([
        x1.astype(jnp.int32) + pos1,
        x2.astype(jnp.int32) + pos2 + _L * _V1,
        x3.astype(jnp.int32) + pos3 + _L * (_V1 + _V2),
        jnp.full((_B, _NI - 3 * _L), nrows, jnp.int32),
    ], axis=1).reshape(_B * _NI)
    h1 = _bag(idx, tf)
    return _mlp(h1, b1, W2, b2, W3, b3)
